# baseline (speedup 1.0000x reference)
import functools

import jax
import jax.numpy as jnp
from jax import lax
from jax.experimental import pallas as pl
from jax.experimental.pallas import tpu as pltpu
from jax.experimental.pallas import tpu_sc as plsc

_B = 4096
_S = 20
_N = _B * _S
_D = 1000
_NC, _NS = 2, 16
_NW = _NC * _NS
_PER_W = _N // _NW
_BPC = 2
_K = _BPC * _S
_CHUNKS = _PER_W // _K
_NBUF = 3


def _sc_gather(x_flat, table):
    mesh = plsc.VectorSubcoreMesh(core_axis_name="c", subcore_axis_name="s")

    @functools.partial(
        pl.kernel,
        mesh=mesh,
        out_type=jax.ShapeDtypeStruct((_B, _S, _D), jnp.float32),
        compiler_params=pltpu.CompilerParams(use_tc_tiling_on_sc=False),
        scratch_types=[
            pltpu.VMEM((_PER_W,), jnp.int32),
            pltpu.VMEM((_NBUF, _K, _D), jnp.float32),
            pltpu.SemaphoreType.DMA,
            pltpu.SemaphoreType.DMA,
        ],
    )
    def k(idx_hbm, table_hbm, out_hbm, idx_v, rows_v, gsem, wsem):
        wid = lax.axis_index("s") * _NC + lax.axis_index("c")
        base = wid * _PER_W
        bbase = wid * (_PER_W // _S)

        pltpu.sync_copy(idx_hbm.at[pl.ds(base, _PER_W)], idx_v)

        def fire(g, slot):
            pltpu.async_copy(table_hbm.at[idx_v.at[pl.ds(g * _K, _K)]],
                             rows_v.at[slot], gsem)

        def wait_gather(g, slot):
            pltpu.make_async_copy(table_hbm.at[idx_v.at[pl.ds(g * _K, _K)]],
                                  rows_v.at[slot], gsem).wait()

        def issue_write(g, slot):
            for b in range(_BPC):
                pltpu.async_copy(rows_v.at[slot, pl.ds(b * _S, _S)],
                                 out_hbm.at[bbase + g * _BPC + b], wsem)

        def wait_write(g, slot):
            for b in range(_BPC):
                pltpu.make_async_copy(rows_v.at[slot, pl.ds(b * _S, _S)],
                                      out_hbm.at[bbase + g * _BPC + b],
                                      wsem).wait()

        for c in range(_NBUF - 1):
            fire(c, c)

        def body(g, _):
            slot = lax.rem(g, _NBUF)

            @pl.when(g >= 1)
            def _():
                wait_write(g - 1, lax.rem(g - 1, _NBUF))

            @pl.when(g + _NBUF - 1 < _CHUNKS)
            def _():
                fire(g + _NBUF - 1, lax.rem(g + _NBUF - 1, _NBUF))

            wait_gather(g, slot)
            issue_write(g, slot)
            return 0

        lax.fori_loop(0, _CHUNKS, body, 0)

        wait_write(_CHUNKS - 1, lax.rem(_CHUNKS - 1, _NBUF))

    return k(x_flat, table)


def kernel(x, table):
    xf = x.reshape(-1).astype(jnp.int32)
    return _sc_gather(xf, table)


# speedup vs baseline: 1.2890x; 1.2890x over previous
import functools

import jax
import jax.numpy as jnp
from jax import lax
from jax.experimental import pallas as pl
from jax.experimental.pallas import tpu as pltpu
from jax.experimental.pallas import tpu_sc as plsc

_B = 4096
_S = 20
_SP = 24
_D = 1000
_DP = 1024
_LB = _DP // 128
_NC, _NS = 2, 16
_NW = _NC * _NS
_BW = _B // _NW
_IPW = _BW * _LB * _SP
_NBUF = 3


def _sc_gather(idx8p, table_r):
    mesh = plsc.VectorSubcoreMesh(core_axis_name="c", subcore_axis_name="s")

    @functools.partial(
        pl.kernel,
        mesh=mesh,
        out_type=jax.ShapeDtypeStruct((_B, _S, _D), jnp.float32),
        scratch_types=[
            pltpu.VMEM((_IPW,), jnp.int32),
            pltpu.VMEM((_NBUF, _S, _D), jnp.float32),
            pltpu.VMEM((_NBUF, _S, 128), jnp.float32),
            pltpu.SemaphoreType.DMA,
            pltpu.SemaphoreType.DMA,
        ],
    )
    def k(idx_hbm, table_hbm, out_hbm, idx_v, cells_v, tail_v, gsem, wsem):
        wid = lax.axis_index("s") * _NC + lax.axis_index("c")
        bbase = wid * _BW

        pltpu.sync_copy(idx_hbm.at[pl.ds(wid * _IPW, _IPW)], idx_v)

        def _gather_dst(slot, lt):
            if lt < _LB - 1:
                return cells_v.at[slot, slice(None), pl.ds(lt * 128, 128)]
            return tail_v.at[slot]

        def fire(g, slot):
            for lt in range(_LB):
                pltpu.async_copy(
                    table_hbm.at[idx_v.at[pl.ds((g * _LB + lt) * _SP, _S)]],
                    _gather_dst(slot, lt), gsem)

        def wait_gather(g, slot):
            for lt in range(_LB):
                pltpu.make_async_copy(
                    table_hbm.at[idx_v.at[pl.ds((g * _LB + lt) * _SP, _S)]],
                    _gather_dst(slot, lt), gsem).wait()

        def patch_tail(slot):
            for r in range(_S):
                for c in range(6):
                    cells_v[slot, r, pl.ds(896 + c * 16, 16)] = (
                        tail_v[slot, r, pl.ds(c * 16, 16)])
                cells_v[slot, r, pl.ds(984, 16)] = (
                    tail_v[slot, r, pl.ds(88, 16)])

        def issue_write(g, slot):
            pltpu.async_copy(cells_v.at[slot], out_hbm.at[bbase + g], wsem)

        def wait_write(g, slot):
            pltpu.make_async_copy(cells_v.at[slot], out_hbm.at[bbase + g],
                                  wsem).wait()

        for c in range(_NBUF - 1):
            fire(c, c)

        def body(g, _):
            slot = lax.rem(g, _NBUF)

            @pl.when(g >= 1)
            def _():
                wait_write(g - 1, lax.rem(g - 1, _NBUF))

            @pl.when(g + _NBUF - 1 < _BW)
            def _():
                fire(g + _NBUF - 1, lax.rem(g + _NBUF - 1, _NBUF))

            wait_gather(g, slot)
            patch_tail(slot)
            issue_write(g, slot)
            return 0

        lax.fori_loop(0, _BW, body, 0)

        wait_write(_BW - 1, lax.rem(_BW - 1, _NBUF))

    return k(idx8p, table_r)


def kernel(x, table):
    xi = x.astype(jnp.int32)
    idx8 = xi[:, None, :] * _LB + jnp.arange(_LB, dtype=jnp.int32)[None, :, None]
    idx8p = jnp.pad(idx8, ((0, 0), (0, 0), (0, _SP - _S))).reshape(-1)
    table_r = jnp.pad(table, ((0, 0), (0, _DP - _D))).reshape(-1, 128)
    return _sc_gather(idx8p, table_r)
